# Initial kernel scaffold; baseline (speedup 1.0000x reference)
import jax
import jax.numpy as jnp
from jax.experimental import pallas as pl


def kernel(logits, top_k):
    raise NotImplementedError("write your pallas kernel here")


# speedup vs baseline: 3.6744x; 3.6744x over previous
import numpy as np
import jax
import jax.numpy as jnp
from jax.experimental import pallas as pl

S = 500
CAP = 64
_TINY = np.float32(np.finfo(np.float32).tiny)
_NEG = np.float32(-np.inf)


def _f32_sort_key(x):
    b = jax.lax.bitcast_convert_type(x, jnp.uint32)
    neg = b >= np.uint32(0x80000000)
    return jnp.where(neg, ~b, b + np.uint32(0x80000000))


def _key_to_f32(u):
    neg = u < np.uint32(0x80000000)
    bits = jnp.where(neg, ~u, u - np.uint32(0x80000000))
    return jax.lax.bitcast_convert_type(bits, jnp.float32)


def _kth_largest_key(keys, kf):
    cur = jnp.uint32(0)
    for b in range(31, -1, -1):
        trial = cur | np.uint32(1 << b)
        cnt = jnp.sum(jnp.where(keys >= trial, np.float32(1.0), np.float32(0.0)))
        cur = jnp.where(cnt >= kf, trial, cur)
    return cur


def _threefry_bits(flat_u32):
    ks0 = np.uint32(0)
    ks1 = np.uint32(42)
    ks2 = np.uint32(0x1BD11BDA) ^ ks1
    keys = [ks0, ks1, ks2]
    rots = [(13, 15, 26, 6), (17, 29, 16, 24)]
    x0 = jnp.zeros_like(flat_u32) + ks0
    x1 = flat_u32 + ks1
    for i in range(5):
        for r in rots[i % 2]:
            x0 = x0 + x1
            x1 = (x1 << np.uint32(r)) | (x1 >> np.uint32(32 - r))
            x1 = x1 ^ x0
        x0 = x0 + keys[(i + 1) % 3]
        x1 = x1 + keys[(i + 2) % 3] + np.uint32(i + 1)
    return x0 ^ x1


def _row_kernel(x_ref, tk_ref, probs_ref, idx_ref, *, V, G, k_sta):
    r = pl.program_id(0)
    kf = jnp.minimum(tk_ref[0, 0], np.float32(k_sta))
    ls = x_ref[0] / np.float32(0.8)
    gmax = jnp.max(ls, axis=1)

    gkey = _f32_sort_key(gmax)
    gstar = _kth_largest_key(gkey, kf)
    maskf = jnp.where(gkey >= gstar, np.float32(1.0), np.float32(0.0))

    GP = 2048
    v = jnp.concatenate([maskf, jnp.zeros((GP - G,), jnp.float32)])
    sh = 1
    while sh < GP:
        v = v + jnp.concatenate([jnp.zeros((sh,), jnp.float32), v[: GP - sh]])
        sh *= 2
    cum = v[:G]
    rank = (cum - maskf).astype(jnp.int32)
    c_total = jnp.minimum(jnp.sum(maskf), np.float32(CAP)).astype(jnp.int32)
    slot = jnp.where(maskf > 0, rank, jnp.int32(-1))

    iota_cap = jax.lax.broadcasted_iota(jnp.int32, (CAP, G), 0)
    onehot = jnp.where(
        (iota_cap == slot[None, :]) & (slot[None, :] >= 0),
        np.float32(1.0), np.float32(0.0))

    cand = jax.lax.dot(onehot, ls, preferred_element_type=jnp.float32,
                       precision=jax.lax.Precision.HIGHEST)
    iota_g = jax.lax.broadcasted_iota(jnp.int32, (G, 1), 0).astype(jnp.float32)
    gidf = jax.lax.dot(onehot, iota_g, preferred_element_type=jnp.float32,
                       precision=jax.lax.Precision.HIGHEST)
    rowvalid = jax.lax.broadcasted_iota(jnp.int32, (CAP, 1), 0) < c_total
    cand = jnp.where(rowvalid, cand, _NEG)

    ckey = _f32_sort_key(cand)
    tstar = _kth_largest_key(ckey, kf)
    thresh = _key_to_f32(tstar)
    vmax = jnp.max(cand)
    kept = cand >= thresh
    total = jnp.sum(jnp.where(kept, jnp.exp(cand - vmax), np.float32(0.0)))

    probs_ref[0] = jnp.where(
        ls >= thresh, jnp.exp(ls - vmax) / total, np.float32(0.0))

    gid = gidf.astype(jnp.int32)
    colpos = gid * S + jax.lax.broadcasted_iota(jnp.int32, (CAP, S), 1)
    flat = (colpos + r * V).astype(jnp.uint32)
    bits = _threefry_bits(flat)
    fl = jax.lax.bitcast_convert_type(
        (bits >> np.uint32(9)) | np.uint32(0x3F800000), jnp.float32)
    fl = fl - np.float32(1.0)
    u = jnp.maximum(_TINY, fl * (np.float32(1.0) - _TINY) + _TINY)
    gumb = -jnp.log(-jnp.log(u))
    score = jnp.where(kept, cand + gumb, _NEG)
    m = jnp.max(score)
    win = jnp.min(jnp.where(score == m, colpos, np.int32(2**30)))
    idx_ref[0, 0] = jnp.full((128,), win, jnp.int32)


def kernel(logits, top_k):
    B, V = logits.shape
    assert V % S == 0
    G = V // S
    k_sta = min(50, V)
    x3 = logits.reshape(B, G, S)
    tk = jnp.broadcast_to(
        jnp.asarray(top_k, jnp.float32).reshape(()), (1, 128))
    probs3, idx3 = pl.pallas_call(
        lambda x_ref, tk_ref, p_ref, i_ref: _row_kernel(
            x_ref, tk_ref, p_ref, i_ref, V=V, G=G, k_sta=k_sta),
        grid=(B,),
        in_specs=[
            pl.BlockSpec((1, G, S), lambda r: (r, 0, 0)),
            pl.BlockSpec((1, 128), lambda r: (0, 0)),
        ],
        out_specs=[
            pl.BlockSpec((1, G, S), lambda r: (r, 0, 0)),
            pl.BlockSpec((1, 1, 128), lambda r: (r, 0, 0)),
        ],
        out_shape=[
            jax.ShapeDtypeStruct((B, G, S), jnp.float32),
            jax.ShapeDtypeStruct((B, 1, 128), jnp.int32),
        ],
    )(x3, tk)
    return probs3.reshape(B, V), idx3[:, 0, 0]


# speedup vs baseline: 3.8069x; 1.0361x over previous
import numpy as np
import jax
import jax.numpy as jnp
from jax.experimental import pallas as pl

S = 500
CAP = 64
_TINY = np.float32(np.finfo(np.float32).tiny)
_NEG = np.float32(-np.inf)


def _f32_sort_key(x):
    b = jax.lax.bitcast_convert_type(x, jnp.uint32)
    neg = b >= np.uint32(0x80000000)
    return jnp.where(neg, ~b, b + np.uint32(0x80000000))


def _key_to_f32(u):
    neg = u < np.uint32(0x80000000)
    bits = jnp.where(neg, ~u, u - np.uint32(0x80000000))
    return jax.lax.bitcast_convert_type(bits, jnp.float32)


def _kth_largest_key(keys, kf):
    cur = jnp.uint32(0)
    one = np.float32(1.0)
    zero = np.float32(0.0)
    for lev in range(15, -1, -1):
        t1 = cur | np.uint32(1 << (2 * lev))
        t2 = cur | np.uint32(2 << (2 * lev))
        t3 = cur | np.uint32(3 << (2 * lev))
        c1 = jnp.sum(jnp.where(keys >= t1, one, zero))
        c2 = jnp.sum(jnp.where(keys >= t2, one, zero))
        c3 = jnp.sum(jnp.where(keys >= t3, one, zero))
        cur = jnp.where(c3 >= kf, t3,
                        jnp.where(c2 >= kf, t2,
                                  jnp.where(c1 >= kf, t1, cur)))
    return cur


def _threefry_bits(flat_u32):
    ks0 = np.uint32(0)
    ks1 = np.uint32(42)
    ks2 = np.uint32(0x1BD11BDA) ^ ks1
    keys = [ks0, ks1, ks2]
    rots = [(13, 15, 26, 6), (17, 29, 16, 24)]
    x0 = jnp.zeros_like(flat_u32) + ks0
    x1 = flat_u32 + ks1
    for i in range(5):
        for r in rots[i % 2]:
            x0 = x0 + x1
            x1 = (x1 << np.uint32(r)) | (x1 >> np.uint32(32 - r))
            x1 = x1 ^ x0
        x0 = x0 + keys[(i + 1) % 3]
        x1 = x1 + keys[(i + 2) % 3] + np.uint32(i + 1)
    return x0 ^ x1


def _row_kernel(x_ref, tk_ref, probs_ref, idx_ref, *, V, G, k_sta):
    r = pl.program_id(0)
    kf = jnp.minimum(tk_ref[0, 0], np.float32(k_sta))
    ls = x_ref[0] / np.float32(0.8)
    gmax = jnp.max(ls, axis=1)

    gkey = _f32_sort_key(gmax)
    gstar = _kth_largest_key(gkey, kf)
    maskf = jnp.where(gkey >= gstar, np.float32(1.0), np.float32(0.0))

    GP = 2048
    v = jnp.concatenate([maskf, jnp.zeros((GP - G,), jnp.float32)])
    sh = 1
    while sh < GP:
        v = v + jnp.concatenate([jnp.zeros((sh,), jnp.float32), v[: GP - sh]])
        sh *= 2
    cum = v[:G]
    rank = (cum - maskf).astype(jnp.int32)
    c_total = jnp.minimum(jnp.sum(maskf), np.float32(CAP)).astype(jnp.int32)
    slot = jnp.where(maskf > 0, rank, jnp.int32(-1))

    iota_cap = jax.lax.broadcasted_iota(jnp.int32, (CAP, G), 0)
    onehot = jnp.where(
        (iota_cap == slot[None, :]) & (slot[None, :] >= 0),
        np.float32(1.0), np.float32(0.0))

    iota_g = jax.lax.broadcasted_iota(jnp.int32, (G, 1), 0).astype(jnp.float32)
    ls_aug = jnp.concatenate([ls, iota_g], axis=1)
    cand_aug = jax.lax.dot(onehot, ls_aug, preferred_element_type=jnp.float32,
                           precision=jax.lax.Precision.HIGHEST)
    cand = cand_aug[:, :S]
    gidf = cand_aug[:, S:]
    rowvalid = jax.lax.broadcasted_iota(jnp.int32, (CAP, 1), 0) < c_total
    cand = jnp.where(rowvalid, cand, _NEG)

    ckey = _f32_sort_key(cand)
    tstar = _kth_largest_key(ckey, kf)
    thresh = _key_to_f32(tstar)
    vmax = jnp.max(cand)
    kept = cand >= thresh
    total = jnp.sum(jnp.where(kept, jnp.exp(cand - vmax), np.float32(0.0)))

    probs_ref[0] = jnp.where(
        ls >= thresh, jnp.exp(ls - vmax) / total, np.float32(0.0))

    gid = gidf.astype(jnp.int32)
    colpos = gid * S + jax.lax.broadcasted_iota(jnp.int32, (CAP, S), 1)
    flat = (colpos + r * V).astype(jnp.uint32)
    bits = _threefry_bits(flat)
    fl = jax.lax.bitcast_convert_type(
        (bits >> np.uint32(9)) | np.uint32(0x3F800000), jnp.float32)
    fl = fl - np.float32(1.0)
    u = jnp.maximum(_TINY, fl * (np.float32(1.0) - _TINY) + _TINY)
    gumb = -jnp.log(-jnp.log(u))
    score = jnp.where(kept, cand + gumb, _NEG)
    m = jnp.max(score)
    win = jnp.min(jnp.where(score == m, colpos, np.int32(2**30)))
    idx_ref[0, 0] = jnp.full((128,), win, jnp.int32)


def kernel(logits, top_k):
    B, V = logits.shape
    assert V % S == 0
    G = V // S
    k_sta = min(50, V)
    x3 = logits.reshape(B, G, S)
    tk = jnp.broadcast_to(
        jnp.asarray(top_k, jnp.float32).reshape(()), (1, 128))
    probs3, idx3 = pl.pallas_call(
        lambda x_ref, tk_ref, p_ref, i_ref: _row_kernel(
            x_ref, tk_ref, p_ref, i_ref, V=V, G=G, k_sta=k_sta),
        grid=(B,),
        in_specs=[
            pl.BlockSpec((1, G, S), lambda r: (r, 0, 0)),
            pl.BlockSpec((1, 128), lambda r: (0, 0)),
        ],
        out_specs=[
            pl.BlockSpec((1, G, S), lambda r: (r, 0, 0)),
            pl.BlockSpec((1, 1, 128), lambda r: (r, 0, 0)),
        ],
        out_shape=[
            jax.ShapeDtypeStruct((B, G, S), jnp.float32),
            jax.ShapeDtypeStruct((B, 1, 128), jnp.int32),
        ],
    )(x3, tk)
    return probs3.reshape(B, V), idx3[:, 0, 0]
